# Initial kernel scaffold; baseline (speedup 1.0000x reference)
import jax
import jax.numpy as jnp
from jax.experimental import pallas as pl


def kernel(x, conv1_w, conv1_b, conv2_w, conv2_b, fc1_w, fc1_b, fc2_w, fc2_b):
    raise NotImplementedError("write your pallas kernel here")


# speedup vs baseline: 3.2028x; 3.2028x over previous
import jax
import jax.numpy as jnp
from jax.experimental import pallas as pl
from jax.experimental.pallas import tpu as pltpu

TN = 128


def _fused_kernel(x_ref, bc1_ref, b1_ref, bc2_ref, b2_ref,
                  w3_ref, b3_ref, w4_ref, b4_ref, out_ref):
    x = x_ref[...]
    bc1 = bc1_ref[...]
    b1 = b1_ref[...]

    def conv1_row(oh):
        slab = x[oh:oh + 5].reshape(160, TN)
        z = jnp.dot(bc1, slab, preferred_element_type=jnp.float32)
        return jnp.maximum(z + b1, 0.0)

    p1_rows = []
    for t in range(12):
        m = jnp.maximum(conv1_row(2 * t), conv1_row(2 * t + 1))
        m = m.reshape(10, 12, 2, TN)
        p1_rows.append(jnp.max(m, axis=2).reshape(120, TN))
    p1 = jnp.stack(p1_rows, axis=0)

    bc2 = bc2_ref[...]
    b2 = b2_ref[...]

    def conv2_row(oh2):
        slab = p1[oh2:oh2 + 5].reshape(600, TN)
        z = jnp.dot(bc2, slab, preferred_element_type=jnp.float32)
        return jnp.maximum(z + b2, 0.0)

    p2_rows = []
    for t in range(4):
        m = jnp.maximum(conv2_row(2 * t), conv2_row(2 * t + 1))
        m = m.reshape(20, 4, 2, TN)
        p2_rows.append(jnp.max(m, axis=2).reshape(80, TN))
    p2 = jnp.stack(p2_rows, axis=0)

    act = p2.reshape(320, TN)
    h1 = jnp.dot(w3_ref[...], act, preferred_element_type=jnp.float32)
    h1 = jnp.maximum(h1 + b3_ref[...], 0.0)

    logits = jnp.dot(w4_ref[...], h1,
                     preferred_element_type=jnp.float32) + b4_ref[...]
    m = jnp.max(logits, axis=0, keepdims=True)
    s = logits - m
    lse = jnp.log(jnp.sum(jnp.exp(s), axis=0, keepdims=True))
    out_ref[...] = s - lse


def kernel(x, conv1_w, conv1_b, conv2_w, conv2_b, fc1_w, fc1_b, fc2_w, fc2_b):
    N, C, H, W = x.shape
    assert (C, H, W) == (1, 28, 28), "Net requires 1x28x28 inputs"
    npad = ((N + TN - 1) // TN) * TN

    xt = x.reshape(N, 28, 28).transpose(1, 2, 0)
    xt = jnp.pad(xt, ((0, 0), (0, 4), (0, npad - N)))

    w1r = conv1_w.reshape(10, 5, 5)
    e1 = (jnp.arange(32)[None, :, None]
          == jnp.arange(24)[:, None, None] + jnp.arange(5)[None, None, :])
    bc1 = jnp.einsum("xwk,cik->cxiw", e1.astype(jnp.float32), w1r)
    bc1 = bc1.reshape(240, 160)
    b1r = jnp.broadcast_to(conv1_b[:, None], (10, 24)).reshape(240, 1)

    e2 = (jnp.arange(12)[None, :, None]
          == jnp.arange(8)[:, None, None] + jnp.arange(5)[None, None, :])
    bc2 = jnp.einsum("xwk,oihk->oxhiw", e2.astype(jnp.float32), conv2_w)
    bc2 = bc2.reshape(160, 600)
    b2r = jnp.broadcast_to(conv2_b[:, None], (20, 8)).reshape(160, 1)

    w3p = fc1_w.reshape(300, 20, 4, 4).transpose(0, 2, 1, 3).reshape(300, 320)
    b3c = fc1_b.reshape(300, 1)
    b4c = fc2_b.reshape(10, 1)

    out = pl.pallas_call(
        _fused_kernel,
        out_shape=jax.ShapeDtypeStruct((10, npad), jnp.float32),
        grid=(npad // TN,),
        in_specs=[
            pl.BlockSpec((28, 32, TN), lambda b: (0, 0, b)),
            pl.BlockSpec((240, 160), lambda b: (0, 0)),
            pl.BlockSpec((240, 1), lambda b: (0, 0)),
            pl.BlockSpec((160, 600), lambda b: (0, 0)),
            pl.BlockSpec((160, 1), lambda b: (0, 0)),
            pl.BlockSpec((300, 320), lambda b: (0, 0)),
            pl.BlockSpec((300, 1), lambda b: (0, 0)),
            pl.BlockSpec((10, 300), lambda b: (0, 0)),
            pl.BlockSpec((10, 1), lambda b: (0, 0)),
        ],
        out_specs=pl.BlockSpec((10, TN), lambda b: (0, b)),
        compiler_params=pltpu.CompilerParams(
            dimension_semantics=("parallel",),
            vmem_limit_bytes=40 * 1024 * 1024,
        ),
    )(xt, bc1, b1r, bc2, b2r, w3p, b3c, fc2_w, b4c)

    return out[:, :N].T


# speedup vs baseline: 5.2688x; 1.6451x over previous
import jax
import jax.numpy as jnp
from jax.experimental import pallas as pl
from jax.experimental.pallas import tpu as pltpu

TN = 128


def _fused_kernel(x_ref, bc1_ref, b1_ref, bc2_ref, b2_ref,
                  w3_ref, b3_ref, w4_ref, b4_ref, out_ref):
    x = x_ref[...]
    bc1e = bc1_ref[0]
    bc1o = bc1_ref[1]
    b1 = b1_ref[...]

    def slab1(oh):
        return x[oh:oh + 5].reshape(160, TN)

    def dot1(b, s):
        return jnp.dot(b, s, preferred_element_type=jnp.float32)

    p1_rows = []
    for t in range(12):
        sa, sb = slab1(2 * t), slab1(2 * t + 1)
        m = jnp.maximum(jnp.maximum(dot1(bc1e, sa), dot1(bc1o, sa)),
                        jnp.maximum(dot1(bc1e, sb), dot1(bc1o, sb)))
        p1_rows.append(jnp.maximum(m + b1, 0.0))
    p1 = jnp.stack(p1_rows, axis=0)

    bc2e = bc2_ref[0]
    bc2o = bc2_ref[1]
    b2 = b2_ref[...]

    p2_rows = []
    for t in range(4):
        sa = p1[2 * t:2 * t + 5].reshape(600, TN)
        sb = p1[2 * t + 1:2 * t + 6].reshape(600, TN)
        m = jnp.maximum(jnp.maximum(dot1(bc2e, sa), dot1(bc2o, sa)),
                        jnp.maximum(dot1(bc2e, sb), dot1(bc2o, sb)))
        p2_rows.append(jnp.maximum(m + b2, 0.0))
    p2 = jnp.stack(p2_rows, axis=0)

    act = p2.reshape(320, TN)
    h1 = jnp.dot(w3_ref[...], act, preferred_element_type=jnp.float32)
    h1 = jnp.maximum(h1 + b3_ref[...], 0.0)

    logits = jnp.dot(w4_ref[...], h1,
                     preferred_element_type=jnp.float32) + b4_ref[...]
    m = jnp.max(logits, axis=0, keepdims=True)
    s = logits - m
    lse = jnp.log(jnp.sum(jnp.exp(s), axis=0, keepdims=True))
    out_ref[...] = s - lse


def kernel(x, conv1_w, conv1_b, conv2_w, conv2_b, fc1_w, fc1_b, fc2_w, fc2_b):
    N, C, H, W = x.shape
    assert (C, H, W) == (1, 28, 28), "Net requires 1x28x28 inputs"
    npad = ((N + TN - 1) // TN) * TN

    xt = x.reshape(N, 28, 28).transpose(1, 2, 0)
    xt = jnp.pad(xt, ((0, 0), (0, 4), (0, npad - N)))

    w1r = conv1_w.reshape(10, 5, 5)

    def banded(n_out, n_w, par, wt, spec):
        e = (jnp.arange(n_w)[None, :, None]
             == 2 * jnp.arange(n_out)[:, None, None]
             + jnp.arange(5)[None, None, :] + par)
        return jnp.einsum(spec, e.astype(jnp.float32), wt)

    bc1 = jnp.stack([banded(12, 32, p, w1r, "xwk,cik->cxiw").reshape(120, 160)
                     for p in (0, 1)])
    b1r = jnp.broadcast_to(conv1_b[:, None], (10, 12)).reshape(120, 1)

    bc2 = jnp.stack([banded(4, 12, p, conv2_w, "xwk,oihk->oxhiw").reshape(80, 600)
                     for p in (0, 1)])
    b2r = jnp.broadcast_to(conv2_b[:, None], (20, 4)).reshape(80, 1)

    w3p = fc1_w.reshape(300, 20, 4, 4).transpose(0, 2, 1, 3).reshape(300, 320)
    b3c = fc1_b.reshape(300, 1)
    b4c = fc2_b.reshape(10, 1)

    out = pl.pallas_call(
        _fused_kernel,
        out_shape=jax.ShapeDtypeStruct((10, npad), jnp.float32),
        grid=(npad // TN,),
        in_specs=[
            pl.BlockSpec((28, 32, TN), lambda b: (0, 0, b)),
            pl.BlockSpec((2, 120, 160), lambda b: (0, 0, 0)),
            pl.BlockSpec((120, 1), lambda b: (0, 0)),
            pl.BlockSpec((2, 80, 600), lambda b: (0, 0, 0)),
            pl.BlockSpec((80, 1), lambda b: (0, 0)),
            pl.BlockSpec((300, 320), lambda b: (0, 0)),
            pl.BlockSpec((300, 1), lambda b: (0, 0)),
            pl.BlockSpec((10, 300), lambda b: (0, 0)),
            pl.BlockSpec((10, 1), lambda b: (0, 0)),
        ],
        out_specs=pl.BlockSpec((10, TN), lambda b: (0, b)),
        compiler_params=pltpu.CompilerParams(
            dimension_semantics=("parallel",),
            vmem_limit_bytes=40 * 1024 * 1024,
        ),
    )(xt, bc1, b1r, bc2, b2r, w3p, b3c, fc2_w, b4c)

    return out[:, :N].T


# speedup vs baseline: 5.7413x; 1.0897x over previous
import jax
import jax.numpy as jnp
from jax.experimental import pallas as pl
from jax.experimental.pallas import tpu as pltpu

TN = 128


def _fused_kernel(x_ref, bc1_ref, b1_ref, bc2_ref, b2_ref,
                  w3_ref, b3_ref, w4_ref, b4_ref, out_ref):
    x = x_ref[...]
    bc1e = bc1_ref[0]
    bc1o = bc1_ref[1]
    b1 = b1_ref[...]

    def slab1(oh):
        return x[oh:oh + 5].reshape(160, TN)

    def dot1(b, s):
        return jnp.dot(b, s, preferred_element_type=jnp.float32)

    zpad = jnp.zeros((8, TN), jnp.bfloat16)
    p1_rows = []
    for t in range(12):
        sa, sb = slab1(2 * t), slab1(2 * t + 1)
        m = jnp.maximum(jnp.maximum(dot1(bc1e, sa), dot1(bc1o, sa)),
                        jnp.maximum(dot1(bc1e, sb), dot1(bc1o, sb)))
        r = jnp.maximum(m + b1, 0.0).astype(jnp.bfloat16)
        p1_rows.append(jnp.concatenate([r, zpad], axis=0))
    p1 = jnp.stack(p1_rows, axis=0)

    bc2e = bc2_ref[0]
    bc2o = bc2_ref[1]
    b2 = b2_ref[...]

    p2_rows = []
    for t in range(4):
        sa = p1[2 * t:2 * t + 5].reshape(640, TN)
        sb = p1[2 * t + 1:2 * t + 6].reshape(640, TN)
        m = jnp.maximum(jnp.maximum(dot1(bc2e, sa), dot1(bc2o, sa)),
                        jnp.maximum(dot1(bc2e, sb), dot1(bc2o, sb)))
        p2_rows.append(jnp.maximum(m + b2, 0.0).astype(jnp.bfloat16))
    p2 = jnp.stack(p2_rows, axis=0)

    act = p2.reshape(320, TN)
    h1 = jnp.dot(w3_ref[...], act, preferred_element_type=jnp.float32)
    h1 = jnp.maximum(h1 + b3_ref[...], 0.0)

    logits = jnp.dot(w4_ref[...], h1,
                     preferred_element_type=jnp.float32) + b4_ref[...]
    m = jnp.max(logits, axis=0, keepdims=True)
    s = logits - m
    lse = jnp.log(jnp.sum(jnp.exp(s), axis=0, keepdims=True))
    out_ref[...] = s - lse


def kernel(x, conv1_w, conv1_b, conv2_w, conv2_b, fc1_w, fc1_b, fc2_w, fc2_b):
    N, C, H, W = x.shape
    assert (C, H, W) == (1, 28, 28), "Net requires 1x28x28 inputs"
    npad = ((N + TN - 1) // TN) * TN

    xt = x.reshape(N, 28, 28).transpose(1, 2, 0).astype(jnp.bfloat16)
    x2 = jnp.pad(xt, ((0, 0), (0, 4), (0, npad - N)))

    w1r = conv1_w.reshape(10, 5, 5)

    def banded(n_out, n_w, par, wt, spec):
        e = (jnp.arange(n_w)[None, :, None]
             == 2 * jnp.arange(n_out)[:, None, None]
             + jnp.arange(5)[None, None, :] + par)
        return jnp.einsum(spec, e.astype(jnp.float32), wt)

    bc1 = jnp.stack([banded(12, 32, p, w1r, "xwk,cik->cxiw").reshape(120, 160)
                     for p in (0, 1)]).astype(jnp.bfloat16)
    b1r = jnp.broadcast_to(conv1_b[:, None], (10, 12)).reshape(120, 1)

    bc2 = jnp.stack(
        [jnp.pad(banded(4, 12, p, conv2_w, "xwk,oihk->oxhiw").reshape(80, 5, 120),
                 ((0, 0), (0, 0), (0, 8))).reshape(80, 640)
         for p in (0, 1)]).astype(jnp.bfloat16)
    b2r = jnp.broadcast_to(conv2_b[:, None], (20, 4)).reshape(80, 1)

    w3p = fc1_w.reshape(300, 20, 4, 4).transpose(0, 2, 1, 3).reshape(300, 320)
    w3p = w3p.astype(jnp.bfloat16)
    b3c = fc1_b.reshape(300, 1)
    b4c = fc2_b.reshape(10, 1)

    out = pl.pallas_call(
        _fused_kernel,
        out_shape=jax.ShapeDtypeStruct((10, npad), jnp.float32),
        grid=(npad // TN,),
        in_specs=[
            pl.BlockSpec((28, 32, TN), lambda b: (0, 0, b)),
            pl.BlockSpec((2, 120, 160), lambda b: (0, 0, 0)),
            pl.BlockSpec((120, 1), lambda b: (0, 0)),
            pl.BlockSpec((2, 80, 640), lambda b: (0, 0, 0)),
            pl.BlockSpec((80, 1), lambda b: (0, 0)),
            pl.BlockSpec((300, 320), lambda b: (0, 0)),
            pl.BlockSpec((300, 1), lambda b: (0, 0)),
            pl.BlockSpec((10, 300), lambda b: (0, 0)),
            pl.BlockSpec((10, 1), lambda b: (0, 0)),
        ],
        out_specs=pl.BlockSpec((10, TN), lambda b: (0, b)),
        compiler_params=pltpu.CompilerParams(
            dimension_semantics=("parallel",),
            vmem_limit_bytes=40 * 1024 * 1024,
        ),
    )(x2, bc1, b1r, bc2, b2r, w3p, b3c, fc2_w, b4c)

    return out[:, :N].T


# speedup vs baseline: 8.1780x; 1.4244x over previous
import jax
import jax.numpy as jnp
from jax.experimental import pallas as pl
from jax.experimental.pallas import tpu as pltpu

TN = 512


def _fused_kernel(x_ref, bc1_ref, b1_ref, bc2_ref, b2_ref,
                  w3_ref, b3_ref, w4_ref, b4_ref, out_ref):
    x = x_ref[...]
    bc1e = bc1_ref[0]
    bc1o = bc1_ref[1]
    b1 = b1_ref[...]

    def slab1(oh):
        return x[oh:oh + 5].reshape(160, TN)

    def dot1(b, s):
        return jnp.dot(b, s, preferred_element_type=jnp.float32)

    zpad = jnp.zeros((8, TN), jnp.bfloat16)
    p1_rows = []
    for t in range(12):
        sa, sb = slab1(2 * t), slab1(2 * t + 1)
        m = jnp.maximum(jnp.maximum(dot1(bc1e, sa), dot1(bc1o, sa)),
                        jnp.maximum(dot1(bc1e, sb), dot1(bc1o, sb)))
        r = jnp.maximum(m + b1, 0.0).astype(jnp.bfloat16)
        p1_rows.append(jnp.concatenate([r, zpad], axis=0))
    p1 = jnp.stack(p1_rows, axis=0)

    bc2e = bc2_ref[0]
    bc2o = bc2_ref[1]
    b2 = b2_ref[...]

    p2_rows = []
    for t in range(4):
        sa = p1[2 * t:2 * t + 5].reshape(640, TN)
        sb = p1[2 * t + 1:2 * t + 6].reshape(640, TN)
        m = jnp.maximum(jnp.maximum(dot1(bc2e, sa), dot1(bc2o, sa)),
                        jnp.maximum(dot1(bc2e, sb), dot1(bc2o, sb)))
        p2_rows.append(jnp.maximum(m + b2, 0.0).astype(jnp.bfloat16))
    p2 = jnp.stack(p2_rows, axis=0)

    act = p2.reshape(320, TN)
    h1 = jnp.dot(w3_ref[...], act, preferred_element_type=jnp.float32)
    h1 = jnp.maximum(h1 + b3_ref[...], 0.0)

    logits = jnp.dot(w4_ref[...], h1,
                     preferred_element_type=jnp.float32) + b4_ref[...]
    m = jnp.max(logits, axis=0, keepdims=True)
    s = logits - m
    lse = jnp.log(jnp.sum(jnp.exp(s), axis=0, keepdims=True))
    out_ref[...] = s - lse


def kernel(x, conv1_w, conv1_b, conv2_w, conv2_b, fc1_w, fc1_b, fc2_w, fc2_b):
    N, C, H, W = x.shape
    assert (C, H, W) == (1, 28, 28), "Net requires 1x28x28 inputs"
    npad = ((N + TN - 1) // TN) * TN

    xt = x.reshape(N, 28, 28).transpose(1, 2, 0).astype(jnp.bfloat16)
    x2 = jnp.pad(xt, ((0, 0), (0, 4), (0, npad - N)))

    w1r = conv1_w.reshape(10, 5, 5)

    def banded(n_out, n_w, par, wt, spec):
        e = (jnp.arange(n_w)[None, :, None]
             == 2 * jnp.arange(n_out)[:, None, None]
             + jnp.arange(5)[None, None, :] + par)
        return jnp.einsum(spec, e.astype(jnp.float32), wt)

    bc1 = jnp.stack([banded(12, 32, p, w1r, "xwk,cik->cxiw").reshape(120, 160)
                     for p in (0, 1)]).astype(jnp.bfloat16)
    b1r = jnp.broadcast_to(conv1_b[:, None], (10, 12)).reshape(120, 1)

    bc2 = jnp.stack(
        [jnp.pad(banded(4, 12, p, conv2_w, "xwk,oihk->oxhiw").reshape(80, 5, 120),
                 ((0, 0), (0, 0), (0, 8))).reshape(80, 640)
         for p in (0, 1)]).astype(jnp.bfloat16)
    b2r = jnp.broadcast_to(conv2_b[:, None], (20, 4)).reshape(80, 1)

    w3p = fc1_w.reshape(300, 20, 4, 4).transpose(0, 2, 1, 3).reshape(300, 320)
    w3p = w3p.astype(jnp.bfloat16)
    b3c = fc1_b.reshape(300, 1)
    b4c = fc2_b.reshape(10, 1)

    out = pl.pallas_call(
        _fused_kernel,
        out_shape=jax.ShapeDtypeStruct((10, npad), jnp.float32),
        grid=(npad // TN,),
        in_specs=[
            pl.BlockSpec((28, 32, TN), lambda b: (0, 0, b)),
            pl.BlockSpec((2, 120, 160), lambda b: (0, 0, 0)),
            pl.BlockSpec((120, 1), lambda b: (0, 0)),
            pl.BlockSpec((2, 80, 640), lambda b: (0, 0, 0)),
            pl.BlockSpec((80, 1), lambda b: (0, 0)),
            pl.BlockSpec((300, 320), lambda b: (0, 0)),
            pl.BlockSpec((300, 1), lambda b: (0, 0)),
            pl.BlockSpec((10, 300), lambda b: (0, 0)),
            pl.BlockSpec((10, 1), lambda b: (0, 0)),
        ],
        out_specs=pl.BlockSpec((10, TN), lambda b: (0, b)),
        compiler_params=pltpu.CompilerParams(
            dimension_semantics=("parallel",),
            vmem_limit_bytes=40 * 1024 * 1024,
        ),
    )(x2, bc1, b1r, bc2, b2r, w3p, b3c, fc2_w, b4c)

    return out[:, :N].T
